# baseline (speedup 1.0000x reference)
import jax
import jax.numpy as jnp
from jax import lax
from jax.experimental import pallas as pl
from jax.experimental.pallas import tpu as pltpu
from jax.experimental.pallas import tpu_sc as plsc

_B = 16384
_N_NUM = 13
_N_CAT = 26
_VOCAB = 100000
_D = 16
_L = 16
_TOK = _N_NUM + _N_CAT

_NC = 2
_NS = 16
_NW = _NC * _NS

_CAT_JOBS_PER_W = (_N_CAT * _D) // _NW
_NUM_JOBS = _N_NUM * _D

_BC = 4096
_NCH = _B // _BC
_GRP = _BC // _L
_NBC = 2048
_NNCH = _B // _NBC
_NGRP = _NBC // _L


def _sc_body(xnum_hbm, xcat_hbm, wflat_hbm, bflat_hbm, tab_hbm, out_hbm,
             tslice, idxb, obuf, nbuf, wv, bv,
             sem_t, sem_i, sem_o0, sem_o1, sem_n0, sem_n1):
    wid = lax.axis_index("s") * _NC + lax.axis_index("c")

    pltpu.sync_copy(wflat_hbm, wv)
    pltpu.sync_copy(bflat_hbm, bv)

    def num_chunk(c):
        fn = lax.div(wid, 2)
        boff = lax.rem(wid, 2) * (_B // 2) + c * _NBC
        xsl = obuf.at[0, pl.ds(0, _NBC)]
        pltpu.sync_copy(xnum_hbm.at[fn, pl.ds(boff, _NBC)], xsl)
        for dd in range(_D):
            sl = dd % 2
            sem_n = sem_n0 if sl == 0 else sem_n1
            if dd >= 2:
                pltpu.make_async_copy(nbuf.at[sl], out_hbm.at[0, 0, pl.ds(0, _NBC)],
                                      sem_n).wait()
            sel = lax.broadcast_in_dim(fn * _D + dd, (_L,), ())
            ws = plsc.load_gather(wv, [sel])
            bs = plsc.load_gather(bv, [sel])

            @plsc.parallel_loop(0, _NGRP, unroll=4)
            def _(g):
                base = g * _L
                nbuf[sl, pl.ds(base, _L)] = obuf[0, pl.ds(base, _L)] * ws + bs

            pltpu.async_copy(nbuf.at[sl], out_hbm.at[fn, dd, pl.ds(boff, _NBC)],
                             sem_n)
        pltpu.make_async_copy(nbuf.at[0], out_hbm.at[0, 0, pl.ds(0, _NBC)],
                              sem_n0).wait()
        pltpu.make_async_copy(nbuf.at[1], out_hbm.at[0, 0, pl.ds(0, _NBC)],
                              sem_n1).wait()

    def cat_iter(i, prev_f):
        j = _CAT_JOBS_PER_W * wid + i
        f = lax.div(j, _D)
        d = lax.rem(j, _D)

        pltpu.async_copy(tab_hbm.at[f, d, :], tslice, sem_t)
        new_f = f != prev_f

        @pl.when(new_f)
        def _():
            pltpu.async_copy(xcat_hbm.at[f, :], idxb, sem_i)

        @pl.when(jnp.logical_and(wid < 2 * _N_NUM, i < (_B // 2) // _NBC))
        def _():
            num_chunk(i)

        @pl.when(new_f)
        def _():
            pltpu.make_async_copy(xcat_hbm.at[f, :], idxb, sem_i).wait()

        pltpu.make_async_copy(tab_hbm.at[f, d, :], tslice, sem_t).wait()

        for c in range(_NCH):
            sl = c % 2
            sem_o = sem_o0 if sl == 0 else sem_o1
            if c >= 2:
                pltpu.make_async_copy(obuf.at[sl], out_hbm.at[0, 0, pl.ds(0, _BC)],
                                      sem_o).wait()
            cbase = c * _BC

            @plsc.parallel_loop(0, _GRP, unroll=4)
            def _(g):
                base = g * _L
                iv = idxb[pl.ds(cbase + base, _L)]
                obuf[sl, pl.ds(base, _L)] = plsc.load_gather(tslice, [iv])

            pltpu.async_copy(obuf.at[sl],
                             out_hbm.at[_N_NUM + f, d, pl.ds(cbase, _BC)], sem_o)
        pltpu.make_async_copy(obuf.at[0], out_hbm.at[0, 0, pl.ds(0, _BC)],
                              sem_o0).wait()
        pltpu.make_async_copy(obuf.at[1], out_hbm.at[0, 0, pl.ds(0, _BC)],
                              sem_o1).wait()
        return f

    lax.fori_loop(0, _CAT_JOBS_PER_W, cat_iter, jnp.int32(-1))


@jax.jit
def _sc_tokenize(xnum_t, xcat_t, wflat, bflat, tab_t):
    mesh = plsc.VectorSubcoreMesh(core_axis_name="c", subcore_axis_name="s")
    k = pl.kernel(
        _sc_body,
        out_type=jax.ShapeDtypeStruct((_TOK, _D, _B), jnp.float32),
        mesh=mesh,
        compiler_params=pltpu.CompilerParams(needs_layout_passes=False),
        scratch_types=[
            pltpu.VMEM((_VOCAB,), jnp.float32),
            pltpu.VMEM((_B,), jnp.int32),
            pltpu.VMEM((2, _BC), jnp.float32),
            pltpu.VMEM((2, _NBC), jnp.float32),
            pltpu.VMEM((_NUM_JOBS,), jnp.float32),
            pltpu.VMEM((_NUM_JOBS,), jnp.float32),
            pltpu.SemaphoreType.DMA,
            pltpu.SemaphoreType.DMA,
            pltpu.SemaphoreType.DMA,
            pltpu.SemaphoreType.DMA,
            pltpu.SemaphoreType.DMA,
            pltpu.SemaphoreType.DMA,
        ],
    )
    return k(xnum_t, xcat_t, wflat, bflat, tab_t)


def kernel(x_num, x_cat, W_num, b_num, tables):
    xnum_t = x_num.T
    xcat_t = x_cat.T.astype(jnp.int32)
    tab_t = jnp.transpose(tables, (0, 2, 1))
    wflat = W_num.reshape(-1)
    bflat = b_num.reshape(-1)
    out_t = _sc_tokenize(xnum_t, xcat_t, wflat, bflat, tab_t)
    return jnp.transpose(out_t, (2, 0, 1))


# speedup vs baseline: 1.0103x; 1.0103x over previous
import jax
import jax.numpy as jnp
from jax import lax
from jax.experimental import pallas as pl
from jax.experimental.pallas import tpu as pltpu
from jax.experimental.pallas import tpu_sc as plsc

_B = 16384
_N_NUM = 13
_N_CAT = 26
_VOCAB = 100000
_D = 16
_L = 16
_TOK = _N_NUM + _N_CAT

_NC = 2
_NS = 16
_NW = _NC * _NS

_CAT_JOBS_PER_W = (_N_CAT * _D) // _NW
_NUM_JOBS = _N_NUM * _D

_BC = 4096
_NCH = _B // _BC
_GRP = _BC // _L
_NBC = 2048
_NNCH = _B // _NBC
_NGRP = _NBC // _L


def _sc_body(xnum_hbm, xcat_hbm, wflat_hbm, bflat_hbm, tab_hbm, out_hbm,
             tslice, idxb, obuf, nbuf, wv, bv,
             sem_t, sem_i, sem_o0, sem_o1, sem_n0, sem_n1):
    wid = lax.axis_index("s") * _NC + lax.axis_index("c")

    pltpu.sync_copy(wflat_hbm, wv)
    pltpu.sync_copy(bflat_hbm, bv)

    def num_chunk(c):
        fn = wid
        boff = c * _NBC
        xsl = obuf.at[0, pl.ds(0, _NBC)]
        pltpu.sync_copy(xnum_hbm.at[fn, pl.ds(boff, _NBC)], xsl)
        for dd in range(_D):
            sl = dd % 2
            sem_n = sem_n0 if sl == 0 else sem_n1
            if dd >= 2:
                pltpu.make_async_copy(nbuf.at[sl], out_hbm.at[0, 0, pl.ds(0, _NBC)],
                                      sem_n).wait()
            sel = lax.broadcast_in_dim(fn * _D + dd, (_L,), ())
            ws = plsc.load_gather(wv, [sel])
            bs = plsc.load_gather(bv, [sel])

            @plsc.parallel_loop(0, _NGRP, unroll=4)
            def _(g):
                base = g * _L
                nbuf[sl, pl.ds(base, _L)] = obuf[0, pl.ds(base, _L)] * ws + bs

            pltpu.async_copy(nbuf.at[sl], out_hbm.at[fn, dd, pl.ds(boff, _NBC)],
                             sem_n)
        pltpu.make_async_copy(nbuf.at[0], out_hbm.at[0, 0, pl.ds(0, _NBC)],
                              sem_n0).wait()
        pltpu.make_async_copy(nbuf.at[1], out_hbm.at[0, 0, pl.ds(0, _NBC)],
                              sem_n1).wait()

    def cat_iter(i, prev_f):
        j = _CAT_JOBS_PER_W * wid + i
        f = lax.div(j, _D)
        d = lax.rem(j, _D)

        pltpu.async_copy(tab_hbm.at[f, d, :], tslice, sem_t)
        new_f = f != prev_f

        @pl.when(new_f)
        def _():
            pltpu.async_copy(xcat_hbm.at[f, :], idxb, sem_i)

        @pl.when(jnp.logical_and(wid < _N_NUM, i < _NNCH))
        def _():
            num_chunk(i)

        @pl.when(new_f)
        def _():
            pltpu.make_async_copy(xcat_hbm.at[f, :], idxb, sem_i).wait()

        pltpu.make_async_copy(tab_hbm.at[f, d, :], tslice, sem_t).wait()

        for c in range(_NCH):
            sl = c % 2
            sem_o = sem_o0 if sl == 0 else sem_o1
            if c >= 2:
                pltpu.make_async_copy(obuf.at[sl], out_hbm.at[0, 0, pl.ds(0, _BC)],
                                      sem_o).wait()
            cbase = c * _BC

            @plsc.parallel_loop(0, _GRP, unroll=4)
            def _(g):
                base = g * _L
                iv = idxb[pl.ds(cbase + base, _L)]
                obuf[sl, pl.ds(base, _L)] = plsc.load_gather(tslice, [iv])

            pltpu.async_copy(obuf.at[sl],
                             out_hbm.at[_N_NUM + f, d, pl.ds(cbase, _BC)], sem_o)
        pltpu.make_async_copy(obuf.at[0], out_hbm.at[0, 0, pl.ds(0, _BC)],
                              sem_o0).wait()
        pltpu.make_async_copy(obuf.at[1], out_hbm.at[0, 0, pl.ds(0, _BC)],
                              sem_o1).wait()
        return f

    lax.fori_loop(0, _CAT_JOBS_PER_W, cat_iter, jnp.int32(-1))


@jax.jit
def _sc_tokenize(xnum_t, xcat_t, wflat, bflat, tab_t):
    mesh = plsc.VectorSubcoreMesh(core_axis_name="c", subcore_axis_name="s")
    k = pl.kernel(
        _sc_body,
        out_type=jax.ShapeDtypeStruct((_TOK, _D, _B), jnp.float32),
        mesh=mesh,
        compiler_params=pltpu.CompilerParams(
            needs_layout_passes=False,
            disable_bounds_checks=True,
            disable_semaphore_checks=True,
        ),
        scratch_types=[
            pltpu.VMEM((_VOCAB,), jnp.float32),
            pltpu.VMEM((_B,), jnp.int32),
            pltpu.VMEM((2, _BC), jnp.float32),
            pltpu.VMEM((2, _NBC), jnp.float32),
            pltpu.VMEM((_NUM_JOBS,), jnp.float32),
            pltpu.VMEM((_NUM_JOBS,), jnp.float32),
            pltpu.SemaphoreType.DMA,
            pltpu.SemaphoreType.DMA,
            pltpu.SemaphoreType.DMA,
            pltpu.SemaphoreType.DMA,
            pltpu.SemaphoreType.DMA,
            pltpu.SemaphoreType.DMA,
        ],
    )
    return k(xnum_t, xcat_t, wflat, bflat, tab_t)


def kernel(x_num, x_cat, W_num, b_num, tables):
    xnum_t = x_num.T
    xcat_t = x_cat.T.astype(jnp.int32)
    tab_t = jnp.transpose(tables, (0, 2, 1))
    wflat = W_num.reshape(-1)
    bflat = b_num.reshape(-1)
    out_t = _sc_tokenize(xnum_t, xcat_t, wflat, bflat, tab_t)
    return jnp.transpose(out_t, (2, 0, 1))
